# baseline (speedup 1.0000x reference)
import functools

import jax
import jax.numpy as jnp
from jax import lax
from jax.experimental import pallas as pl
from jax.experimental.pallas import tpu as pltpu
from jax.experimental.pallas import tpu_sc as plsc

_NEG_INF = 1e30
_F32 = jnp.float32
_BF16 = jnp.bfloat16


def _ln_t_body(x_ref, g_ref, b_ref, o_ref):
    x = x_ref[...]
    mu = jnp.mean(x, axis=0, keepdims=True)
    var = jnp.mean((x - mu) ** 2, axis=0, keepdims=True)
    o_ref[...] = (x - mu) * lax.rsqrt(var + 1e-6) * g_ref[...] + b_ref[...]


def _head_ln_t(x):
    mu = jnp.mean(x, axis=0, keepdims=True)
    var = jnp.mean((x - mu) ** 2, axis=0, keepdims=True)
    return (x - mu) * lax.rsqrt(var + 1e-6)


def _proj_qk_body(xt_ref, wq_ref, wk_ref, u_ref, vb_ref,
                  qu_ref, qv_ref, kn_ref, *, tau):
    xt = xt_ref[...]
    q = lax.dot_general(wq_ref[0], xt, (((1,), (0,)), ((), ())),
                        preferred_element_type=_F32)
    qn = _head_ln_t(q) * (1.0 / tau)
    qu_ref[...] = (qn + u_ref[0]).astype(_BF16)
    qv_ref[...] = (qn + vb_ref[0]).astype(_BF16)
    k = lax.dot_general(wk_ref[0], xt, (((1,), (0,)), ((), ())),
                        preferred_element_type=_F32)
    kn_ref[...] = _head_ln_t(k)


def _proj_vg_body(x_ref, g_ref, b_ref, wv_ref, wg_ref, v_ref, gate_ref, *, h):
    x = x_ref[...]
    mu = jnp.mean(x, axis=-1, keepdims=True)
    var = jnp.mean((x - mu) ** 2, axis=-1, keepdims=True)
    xt = (x - mu) * lax.rsqrt(var + 1e-6) * g_ref[...] + b_ref[...]
    v = jnp.dot(xt, wv_ref[...], preferred_element_type=_F32)
    g = jnp.dot(xt, wg_ref[...], preferred_element_type=_F32)
    gate = g * jax.nn.sigmoid(g)
    dv = v.shape[1] // h
    for i in range(h):
        v_ref[i] = v[:, i * dv:(i + 1) * dv].astype(_BF16)
        gate_ref[i] = gate[:, i * dv:(i + 1) * dv]


def _vq_body(kn_ref, cb_ref, mask_ref, z_ref, loss_ref, cbf_ref, csq2_ref,
             *, n_codes, cs):
    hh = pl.program_id(0)
    ib = pl.program_id(1)
    kt = kn_ref[...]
    lb = kt.shape[1]

    @pl.when(ib == 0)
    def _():
        c = cb_ref[0]
        cbf_ref[...] = c.astype(_BF16)
        csq2_ref[...] = 0.5 * jnp.sum(c * c, axis=1, keepdims=True)

    kbf = kt.astype(_BF16)
    nch = n_codes // cs

    def step(i, carry):
        runmin, runidx = carry
        c0 = i * cs
        dch = lax.dot_general(cbf_ref[pl.ds(c0, cs), :], kbf,
                              (((1,), (0,)), ((), ())),
                              preferred_element_type=_F32)
        d2 = csq2_ref[pl.ds(c0, cs), :] - dch
        cond = d2 < runmin
        return (jnp.where(cond, d2, runmin),
                jnp.where(cond, jnp.broadcast_to(i, runidx.shape), runidx))

    runmin0 = jnp.full((cs, lb), jnp.inf, _F32)
    runidx0 = jnp.zeros((cs, lb), jnp.int32)
    runmin, runidx = lax.fori_loop(0, nch, step, (runmin0, runidx0))

    minv2 = jnp.min(runmin, axis=0, keepdims=True)
    rowg = runidx * cs + lax.broadcasted_iota(jnp.int32, (cs, lb), 0)
    cand = jnp.where(runmin <= minv2, rowg, n_codes)
    z = jnp.min(cand, axis=0, keepdims=True)
    z_ref[0, 0] = z + hh * n_codes

    minv = 2.0 * minv2
    ksq = jnp.sum(kt * kt, axis=0, keepdims=True)
    local = jnp.sum(mask_ref[...] * (ksq + minv), axis=1, keepdims=True)

    @pl.when((hh == 0) & (ib == 0))
    def _():
        loss_ref[...] = jnp.zeros_like(loss_ref)

    loss_ref[...] += local


def _sc_gather(table, idx):
    _, dd = table.shape
    bt = idx.shape[0]
    info = plsc.get_sparse_core_info()
    nw = info.num_cores * info.num_subcores
    b_per_w = bt // nw
    mesh = plsc.VectorSubcoreMesh(core_axis_name="c", subcore_axis_name="s")

    @functools.partial(
        pl.kernel, mesh=mesh,
        out_type=jax.ShapeDtypeStruct((bt, dd), _F32),
        compiler_params=pltpu.CompilerParams(use_tc_tiling_on_sc=False),
        scratch_types=[
            pltpu.VMEM((b_per_w,), jnp.int32),
            pltpu.VMEM((b_per_w, dd), _F32),
            pltpu.SemaphoreType.DMA,
        ],
    )
    def k(table_hbm, idx_hbm, out_hbm, idx_v, rows_v, sem):
        wid = lax.axis_index("s") * info.num_cores + lax.axis_index("c")
        base = wid * b_per_w
        pltpu.sync_copy(idx_hbm.at[pl.ds(base, b_per_w)], idx_v)
        pltpu.async_copy(table_hbm.at[idx_v], rows_v, sem).wait()
        pltpu.sync_copy(rows_v, out_hbm.at[pl.ds(base, b_per_w)])

    return k(table, idx)


def _attn_body(qu_ref, qv_ref, kh_ref, v_ref, pg_ref, wv_ref, *, tm, tn, l):
    ib = pl.program_id(1)
    i0 = ib * tm
    w2 = tm + tn
    dv = wv_ref.shape[2]

    qu = qu_ref[...]
    qv = qv_ref[...]
    rowi = i0 + lax.broadcasted_iota(jnp.int32, (tm, tn), 0)

    def step(jb, carry):
        m, ssum, acc = carry
        j0 = jb * tn
        ac = lax.dot_general(qu, kh_ref[0, :, pl.ds(j0, tn)],
                             (((0,), (0,)), ((), ())),
                             preferred_element_type=_F32)
        t0 = l - tm - i0 + j0
        g = lax.dot_general(qv, pg_ref[:, pl.ds(t0, w2)],
                            (((0,), (0,)), ((), ())),
                            preferred_element_type=_F32)
        rolled = pltpu.roll(g, tn + 1, axis=1, stride=1, stride_axis=0)
        s = ac + rolled[:, :tn]
        coli = j0 + lax.broadcasted_iota(jnp.int32, (tm, tn), 1)
        s = jnp.where(coli <= rowi, s, -_NEG_INF)
        m_new = jnp.maximum(m, jnp.max(s, axis=1, keepdims=True))
        alpha = jnp.exp(m - m_new)
        p = jnp.exp(s - m_new)
        ssum = alpha * ssum + jnp.sum(p, axis=1, keepdims=True)
        pv = lax.dot_general(p.astype(_BF16), v_ref[0, pl.ds(j0, tn), :],
                             (((1,), (0,)), ((), ())),
                             preferred_element_type=_F32)
        return m_new, ssum, alpha * acc + pv

    m0 = jnp.full((tm, 1), -jnp.inf, _F32)
    s0 = jnp.zeros((tm, 1), _F32)
    a0 = jnp.zeros((tm, dv), _F32)
    _, ssum, acc = lax.fori_loop(0, (i0 + tm) // tn, step, (m0, s0, a0))
    wv_ref[0] = acc / ssum


def _out_body(wv_ref, gate_ref, x_ref, wres_ref, o_ref, *, h):
    acc = x_ref[...]
    for i in range(h):
        acc = acc + jnp.dot(wv_ref[i] * gate_ref[i], wres_ref[i],
                            preferred_element_type=_F32)
    o_ref[...] = acc


def kernel(input_features, doc_ids, loss_mask, ln_g, ln_b, W_q, W_k, W_v,
           W_g, W_res, xl_u, xl_v, codebook):
    del doc_ids
    b, l, d = input_features.shape
    h, s, dk = codebook.shape
    dv = W_v.shape[1] // h
    tau = float(dk) ** 0.5
    x2 = input_features.reshape(l, d)
    x2t = x2.T

    wq3 = W_q.reshape(d, h, dk).transpose(1, 2, 0)
    wk3 = W_k.reshape(d, h, dk).transpose(1, 2, 0)
    wres3 = W_res.reshape(h, dv, d)
    u3 = xl_u.reshape(h, dk, 1)
    vb3 = xl_v.reshape(h, dk, 1)
    pos = jnp.arange(l - 1, -l - 1, -1, dtype=_F32)
    inv = 1.0 / (10000.0 ** (jnp.arange(0, dk, 2, dtype=_F32) / dk))
    ang = pos[:, None] * inv[None, :]
    pgt = jnp.concatenate([jnp.sin(ang), jnp.cos(ang)],
                          axis=-1).T.astype(_BF16)
    mask_row = loss_mask.reshape(1, l)

    tb = 512
    xt = pl.pallas_call(
        _ln_t_body,
        grid=(l // tb,),
        in_specs=[
            pl.BlockSpec((d, tb), lambda i: (0, i)),
            pl.BlockSpec((d, 1), lambda i: (0, 0)),
            pl.BlockSpec((d, 1), lambda i: (0, 0)),
        ],
        out_specs=pl.BlockSpec((d, tb), lambda i: (0, i)),
        out_shape=jax.ShapeDtypeStruct((d, l), _F32),
    )(x2t, ln_g.reshape(d, 1), ln_b.reshape(d, 1))

    hblk = lambda hh, ib: (hh, 0, 0)
    tblk = lambda hh, ib: (hh, ib)
    qu, qv, kn = pl.pallas_call(
        functools.partial(_proj_qk_body, tau=tau),
        grid=(h, l // tb),
        in_specs=[
            pl.BlockSpec((d, tb), lambda hh, ib: (0, ib)),
            pl.BlockSpec((1, dk, d), hblk),
            pl.BlockSpec((1, dk, d), hblk),
            pl.BlockSpec((1, dk, 1), hblk),
            pl.BlockSpec((1, dk, 1), hblk),
        ],
        out_specs=[
            pl.BlockSpec((dk, tb), tblk),
            pl.BlockSpec((dk, tb), tblk),
            pl.BlockSpec((dk, tb), tblk),
        ],
        out_shape=[
            jax.ShapeDtypeStruct((h * dk, l), _BF16),
            jax.ShapeDtypeStruct((h * dk, l), _BF16),
            jax.ShapeDtypeStruct((h * dk, l), _F32),
        ],
    )(xt, wq3, wk3, u3, vb3)

    vv, gate = pl.pallas_call(
        functools.partial(_proj_vg_body, h=h),
        grid=(l // tb,),
        in_specs=[
            pl.BlockSpec((tb, d), lambda i: (i, 0)),
            pl.BlockSpec((1, d), lambda i: (0, 0)),
            pl.BlockSpec((1, d), lambda i: (0, 0)),
            pl.BlockSpec((d, h * dv), lambda i: (0, 0)),
            pl.BlockSpec((d, h * dv), lambda i: (0, 0)),
        ],
        out_specs=[
            pl.BlockSpec((h, tb, dv), lambda i: (0, i, 0)),
            pl.BlockSpec((h, tb, dv), lambda i: (0, i, 0)),
        ],
        out_shape=[
            jax.ShapeDtypeStruct((h, l, dv), _BF16),
            jax.ShapeDtypeStruct((h, l, dv), _F32),
        ],
    )(x2, ln_g.reshape(1, d), ln_b.reshape(1, d), W_v, W_g)

    lb = 256
    zidx, lacc = pl.pallas_call(
        functools.partial(_vq_body, n_codes=s, cs=64),
        grid=(h, l // lb),
        scratch_shapes=[
            pltpu.VMEM((s, dk), _BF16),
            pltpu.VMEM((s, 1), _F32),
        ],
        in_specs=[
            pl.BlockSpec((dk, lb), tblk),
            pl.BlockSpec((1, s, dk), hblk),
            pl.BlockSpec((1, lb), lambda hh, ib: (0, ib)),
        ],
        out_specs=[
            pl.BlockSpec((1, 1, 1, lb), lambda hh, ib: (hh, ib, 0, 0)),
            pl.BlockSpec((1, 1), lambda hh, ib: (0, 0)),
        ],
        out_shape=[
            jax.ShapeDtypeStruct((h, l // lb, 1, lb), jnp.int32),
            jax.ShapeDtypeStruct((1, 1), _F32),
        ],
    )(kn, codebook, mask_row)

    khat = _sc_gather(codebook.reshape(h * s, dk), zidx.reshape(h * l))
    khatt = khat.reshape(h, l, dk).transpose(0, 2, 1).astype(_BF16)

    tm = 256
    tn = 256
    wv = pl.pallas_call(
        functools.partial(_attn_body, tm=tm, tn=tn, l=l),
        grid=(h, l // tm),
        in_specs=[
            pl.BlockSpec((dk, tm), tblk),
            pl.BlockSpec((dk, tm), tblk),
            pl.BlockSpec((1, dk, l), hblk),
            pl.BlockSpec((1, l, dv), hblk),
            pl.BlockSpec((dk, 2 * l), lambda hh, ib: (0, 0)),
        ],
        out_specs=pl.BlockSpec((1, tm, dv), lambda hh, ib: (hh, ib, 0)),
        out_shape=jax.ShapeDtypeStruct((h, l, dv), _F32),
    )(qu, qv, khatt, vv, pgt)

    out = pl.pallas_call(
        functools.partial(_out_body, h=h),
        grid=(l // tb,),
        in_specs=[
            pl.BlockSpec((h, tb, dv), lambda i: (0, i, 0)),
            pl.BlockSpec((h, tb, dv), lambda i: (0, i, 0)),
            pl.BlockSpec((tb, d), lambda i: (i, 0)),
            pl.BlockSpec((h, dv, d), lambda i: (0, 0, 0)),
        ],
        out_specs=pl.BlockSpec((tb, d), lambda i: (i, 0)),
        out_shape=jax.ShapeDtypeStruct((l, d), _F32),
    )(wv, gate, x2, wres3)

    loss = lacc[0, 0] / (b * h * l)
    return out.reshape(b, l, d), loss, loss


# speedup vs baseline: 3.1403x; 3.1403x over previous
import functools

import jax
import jax.numpy as jnp
from jax import lax
from jax.experimental import pallas as pl
from jax.experimental.pallas import tpu as pltpu
from jax.experimental.pallas import tpu_sc as plsc

_NEG_INF = 1e30
_F32 = jnp.float32
_BF16 = jnp.bfloat16


def _ln_t_body(x_ref, g_ref, b_ref, o_ref):
    x = x_ref[...]
    mu = jnp.mean(x, axis=0, keepdims=True)
    var = jnp.mean((x - mu) ** 2, axis=0, keepdims=True)
    o_ref[...] = (x - mu) * lax.rsqrt(var + 1e-6) * g_ref[...] + b_ref[...]


def _head_ln_t(x):
    mu = jnp.mean(x, axis=0, keepdims=True)
    var = jnp.mean((x - mu) ** 2, axis=0, keepdims=True)
    return (x - mu) * lax.rsqrt(var + 1e-6)


def _proj_qk_body(xt_ref, wq_ref, wk_ref, u_ref, vb_ref,
                  qu_ref, qv_ref, kn_ref, *, tau):
    xt = xt_ref[...]
    q = lax.dot_general(wq_ref[0], xt, (((1,), (0,)), ((), ())),
                        preferred_element_type=_F32)
    qn = _head_ln_t(q) * (1.0 / tau)
    qu_ref[...] = (qn + u_ref[0]).astype(_BF16)
    qv_ref[...] = (qn + vb_ref[0]).astype(_BF16)
    k = lax.dot_general(wk_ref[0], xt, (((1,), (0,)), ((), ())),
                        preferred_element_type=_F32)
    kn_ref[...] = _head_ln_t(k)


def _proj_vg_body(x_ref, g_ref, b_ref, wv_ref, wg_ref, v_ref, gate_ref, *, h):
    x = x_ref[...]
    mu = jnp.mean(x, axis=-1, keepdims=True)
    var = jnp.mean((x - mu) ** 2, axis=-1, keepdims=True)
    xt = (x - mu) * lax.rsqrt(var + 1e-6) * g_ref[...] + b_ref[...]
    v = jnp.dot(xt, wv_ref[...], preferred_element_type=_F32)
    g = jnp.dot(xt, wg_ref[...], preferred_element_type=_F32)
    gate = g * jax.nn.sigmoid(g)
    dv = v.shape[1] // h
    for i in range(h):
        v_ref[i] = v[:, i * dv:(i + 1) * dv].astype(_BF16)
        gate_ref[i] = gate[:, i * dv:(i + 1) * dv]


def _vq_body(kn_ref, cb_ref, mask_ref, z_ref, loss_ref, cbf_ref, csq2_ref,
             *, n_codes, cs):
    hh = pl.program_id(0)
    ib = pl.program_id(1)
    kt = kn_ref[...]
    lb = kt.shape[1]

    @pl.when(ib == 0)
    def _():
        c = cb_ref[0]
        cbf_ref[...] = c.astype(_BF16)
        csq2_ref[...] = 0.5 * jnp.sum(c * c, axis=1, keepdims=True)

    kbf = kt.astype(_BF16)
    nch = n_codes // cs

    runmin = jnp.full((cs, lb), jnp.inf, _F32)
    runidx = jnp.zeros((cs, lb), jnp.int32)
    for i in range(nch):
        dch = lax.dot_general(cbf_ref[i * cs:(i + 1) * cs, :], kbf,
                              (((1,), (0,)), ((), ())),
                              preferred_element_type=_F32)
        d2 = csq2_ref[i * cs:(i + 1) * cs, :] - dch
        cond = d2 < runmin
        runmin = jnp.where(cond, d2, runmin)
        runidx = jnp.where(cond, jnp.full((cs, lb), i, jnp.int32), runidx)

    minv2 = jnp.min(runmin, axis=0, keepdims=True)
    rowg = runidx * cs + lax.broadcasted_iota(jnp.int32, (cs, lb), 0)
    cand = jnp.where(runmin <= minv2, rowg, n_codes)
    z = jnp.min(cand, axis=0, keepdims=True)
    z_ref[0, 0] = z + hh * n_codes

    minv = 2.0 * minv2
    ksq = jnp.sum(kt * kt, axis=0, keepdims=True)
    local = jnp.sum(mask_ref[...] * (ksq + minv), axis=1, keepdims=True)

    @pl.when((hh == 0) & (ib == 0))
    def _():
        loss_ref[...] = jnp.zeros_like(loss_ref)

    loss_ref[...] += local


def _sc_gather(table, idx):
    _, dd = table.shape
    bt = idx.shape[0]
    info = plsc.get_sparse_core_info()
    nw = info.num_cores * info.num_subcores
    b_per_w = bt // nw
    mesh = plsc.VectorSubcoreMesh(core_axis_name="c", subcore_axis_name="s")

    @functools.partial(
        pl.kernel, mesh=mesh,
        out_type=jax.ShapeDtypeStruct((bt, dd), _F32),
        compiler_params=pltpu.CompilerParams(use_tc_tiling_on_sc=False),
        scratch_types=[
            pltpu.VMEM((b_per_w,), jnp.int32),
            pltpu.VMEM((b_per_w, dd), _F32),
            pltpu.SemaphoreType.DMA,
        ],
    )
    def k(table_hbm, idx_hbm, out_hbm, idx_v, rows_v, sem):
        wid = lax.axis_index("s") * info.num_cores + lax.axis_index("c")
        base = wid * b_per_w
        pltpu.sync_copy(idx_hbm.at[pl.ds(base, b_per_w)], idx_v)
        pltpu.async_copy(table_hbm.at[idx_v], rows_v, sem).wait()
        pltpu.sync_copy(rows_v, out_hbm.at[pl.ds(base, b_per_w)])

    return k(table, idx)


def _attn_body(qu_ref, qv_ref, kh_ref, v_ref, pg_ref, wv_ref,
               m_ref, s_ref, acc_ref, *, tm, tn, l):
    ib = pl.program_id(1)
    i0 = ib * tm
    w2 = tm + tn

    qu = qu_ref[...]
    qv = qv_ref[...]
    rowi = i0 + lax.broadcasted_iota(jnp.int32, (tm, tn), 0)

    m_ref[...] = jnp.full_like(m_ref, -jnp.inf)
    s_ref[...] = jnp.zeros_like(s_ref)
    acc_ref[...] = jnp.zeros_like(acc_ref)

    for jb in range(l // tn):
        @pl.when(jb <= ib)
        def _():
            j0 = jb * tn
            ac = lax.dot_general(qu, kh_ref[0, :, j0:j0 + tn],
                                 (((0,), (0,)), ((), ())),
                                 preferred_element_type=_F32)
            t0 = l - tm - i0 + j0
            g = lax.dot_general(qv, pg_ref[:, pl.ds(t0, w2)],
                                (((0,), (0,)), ((), ())),
                                preferred_element_type=_F32)
            rolled = pltpu.roll(g, tn + 1, axis=1, stride=1, stride_axis=0)
            s = ac + rolled[:, :tn]
            coli = j0 + lax.broadcasted_iota(jnp.int32, (tm, tn), 1)
            s = jnp.where(coli <= rowi, s, -_NEG_INF)
            m = m_ref[...]
            m_new = jnp.maximum(m, jnp.max(s, axis=1, keepdims=True))
            alpha = jnp.exp(m - m_new)
            p = jnp.exp(s - m_new)
            s_ref[...] = alpha * s_ref[...] + jnp.sum(p, axis=1, keepdims=True)
            pv = lax.dot_general(p.astype(_BF16), v_ref[0, j0:j0 + tn, :],
                                 (((1,), (0,)), ((), ())),
                                 preferred_element_type=_F32)
            acc_ref[...] = alpha * acc_ref[...] + pv
            m_ref[...] = m_new

    wv_ref[0] = acc_ref[...] / s_ref[...]


def _out_body(wv_ref, gate_ref, x_ref, wres_ref, o_ref, *, h):
    acc = x_ref[...]
    for i in range(h):
        acc = acc + jnp.dot(wv_ref[i] * gate_ref[i], wres_ref[i],
                            preferred_element_type=_F32)
    o_ref[...] = acc


def kernel(input_features, doc_ids, loss_mask, ln_g, ln_b, W_q, W_k, W_v,
           W_g, W_res, xl_u, xl_v, codebook):
    del doc_ids
    b, l, d = input_features.shape
    h, s, dk = codebook.shape
    dv = W_v.shape[1] // h
    tau = float(dk) ** 0.5
    x2 = input_features.reshape(l, d)
    x2t = x2.T

    wq3 = W_q.reshape(d, h, dk).transpose(1, 2, 0)
    wk3 = W_k.reshape(d, h, dk).transpose(1, 2, 0)
    wres3 = W_res.reshape(h, dv, d)
    u3 = xl_u.reshape(h, dk, 1)
    vb3 = xl_v.reshape(h, dk, 1)
    pos = jnp.arange(l - 1, -l - 1, -1, dtype=_F32)
    inv = 1.0 / (10000.0 ** (jnp.arange(0, dk, 2, dtype=_F32) / dk))
    ang = pos[:, None] * inv[None, :]
    pgt = jnp.concatenate([jnp.sin(ang), jnp.cos(ang)],
                          axis=-1).T.astype(_BF16)
    mask_row = loss_mask.reshape(1, l)

    tb = 512
    xt = pl.pallas_call(
        _ln_t_body,
        grid=(l // tb,),
        in_specs=[
            pl.BlockSpec((d, tb), lambda i: (0, i)),
            pl.BlockSpec((d, 1), lambda i: (0, 0)),
            pl.BlockSpec((d, 1), lambda i: (0, 0)),
        ],
        out_specs=pl.BlockSpec((d, tb), lambda i: (0, i)),
        out_shape=jax.ShapeDtypeStruct((d, l), _F32),
    )(x2t, ln_g.reshape(d, 1), ln_b.reshape(d, 1))

    hblk = lambda hh, ib: (hh, 0, 0)
    tblk = lambda hh, ib: (hh, ib)
    qu, qv, kn = pl.pallas_call(
        functools.partial(_proj_qk_body, tau=tau),
        grid=(h, l // tb),
        in_specs=[
            pl.BlockSpec((d, tb), lambda hh, ib: (0, ib)),
            pl.BlockSpec((1, dk, d), hblk),
            pl.BlockSpec((1, dk, d), hblk),
            pl.BlockSpec((1, dk, 1), hblk),
            pl.BlockSpec((1, dk, 1), hblk),
        ],
        out_specs=[
            pl.BlockSpec((dk, tb), tblk),
            pl.BlockSpec((dk, tb), tblk),
            pl.BlockSpec((dk, tb), tblk),
        ],
        out_shape=[
            jax.ShapeDtypeStruct((h * dk, l), _BF16),
            jax.ShapeDtypeStruct((h * dk, l), _BF16),
            jax.ShapeDtypeStruct((h * dk, l), _F32),
        ],
    )(xt, wq3, wk3, u3, vb3)

    vv, gate = pl.pallas_call(
        functools.partial(_proj_vg_body, h=h),
        grid=(l // tb,),
        in_specs=[
            pl.BlockSpec((tb, d), lambda i: (i, 0)),
            pl.BlockSpec((1, d), lambda i: (0, 0)),
            pl.BlockSpec((1, d), lambda i: (0, 0)),
            pl.BlockSpec((d, h * dv), lambda i: (0, 0)),
            pl.BlockSpec((d, h * dv), lambda i: (0, 0)),
        ],
        out_specs=[
            pl.BlockSpec((h, tb, dv), lambda i: (0, i, 0)),
            pl.BlockSpec((h, tb, dv), lambda i: (0, i, 0)),
        ],
        out_shape=[
            jax.ShapeDtypeStruct((h, l, dv), _BF16),
            jax.ShapeDtypeStruct((h, l, dv), _F32),
        ],
    )(x2, ln_g.reshape(1, d), ln_b.reshape(1, d), W_v, W_g)

    lb = 256
    zidx, lacc = pl.pallas_call(
        functools.partial(_vq_body, n_codes=s, cs=64),
        grid=(h, l // lb),
        scratch_shapes=[
            pltpu.VMEM((s, dk), _BF16),
            pltpu.VMEM((s, 1), _F32),
        ],
        in_specs=[
            pl.BlockSpec((dk, lb), tblk),
            pl.BlockSpec((1, s, dk), hblk),
            pl.BlockSpec((1, lb), lambda hh, ib: (0, ib)),
        ],
        out_specs=[
            pl.BlockSpec((1, 1, 1, lb), lambda hh, ib: (hh, ib, 0, 0)),
            pl.BlockSpec((1, 1), lambda hh, ib: (0, 0)),
        ],
        out_shape=[
            jax.ShapeDtypeStruct((h, l // lb, 1, lb), jnp.int32),
            jax.ShapeDtypeStruct((1, 1), _F32),
        ],
    )(kn, codebook, mask_row)

    khat = _sc_gather(codebook.reshape(h * s, dk), zidx.reshape(h * l))
    khatt = khat.reshape(h, l, dk).transpose(0, 2, 1).astype(_BF16)

    tm = 256
    tn = 256
    wv = pl.pallas_call(
        functools.partial(_attn_body, tm=tm, tn=tn, l=l),
        grid=(h, l // tm),
        scratch_shapes=[
            pltpu.VMEM((tm, 1), _F32),
            pltpu.VMEM((tm, 1), _F32),
            pltpu.VMEM((tm, dv), _F32),
        ],
        in_specs=[
            pl.BlockSpec((dk, tm), tblk),
            pl.BlockSpec((dk, tm), tblk),
            pl.BlockSpec((1, dk, l), hblk),
            pl.BlockSpec((1, l, dv), hblk),
            pl.BlockSpec((dk, 2 * l), lambda hh, ib: (0, 0)),
        ],
        out_specs=pl.BlockSpec((1, tm, dv), lambda hh, ib: (hh, ib, 0)),
        out_shape=jax.ShapeDtypeStruct((h, l, dv), _F32),
    )(qu, qv, khatt, vv, pgt)

    out = pl.pallas_call(
        functools.partial(_out_body, h=h),
        grid=(l // tb,),
        in_specs=[
            pl.BlockSpec((h, tb, dv), lambda i: (0, i, 0)),
            pl.BlockSpec((h, tb, dv), lambda i: (0, i, 0)),
            pl.BlockSpec((tb, d), lambda i: (i, 0)),
            pl.BlockSpec((h, dv, d), lambda i: (0, 0, 0)),
        ],
        out_specs=pl.BlockSpec((tb, d), lambda i: (i, 0)),
        out_shape=jax.ShapeDtypeStruct((l, d), _F32),
    )(wv, gate, x2, wres3)

    loss = lacc[0, 0] / (b * h * l)
    return out.reshape(b, l, d), loss, loss
